# baseline (speedup 1.0000x reference)
import jax
import jax.numpy as jnp
from jax import lax
from jax.experimental import pallas as pl
from jax.experimental.pallas import tpu as pltpu
from jax.experimental.pallas import tpu_sc as plsc

LATENT = 32
BATCH = 16384
TABLE = 1000000

NUM_CORES = 2
NUM_SUBCORES = 16
LANES = 16
NW = NUM_CORES * NUM_SUBCORES
B_PER_W = BATCH // NW

TILECOLS = TABLE // 128
R_PER_W = ((TILECOLS + NW - 1) // NW) * 128
WIN = 512
NWIN = (R_PER_W + WIN - 1) // WIN
ALIGNED_END = TILECOLS * 128
MAX_WSTART = ALIGNED_END - WIN
TAIL = 128
GATH_ROWS = BATCH + 16
FLUSH = 64


def _gather_body(idxu_hbm, idxi_hbm, user_t_hbm, item_t_hbm,
                 utail_hbm, itail_hbm, ugath_hbm, vgath_hbm,
                 idx_v, b_v, wb_v, win_v, tail_v, stage_v, bstage_v,
                 sem, ssem):
    wid = lax.axis_index("s") * NUM_CORES + lax.axis_index("c")
    lo = wid * R_PER_W
    hi = jnp.minimum(lo + R_PER_W, TABLE)

    lane = lax.iota(jnp.int32, LANES)
    rows_lo = lane
    rows_hi = lane + LANES

    def reset_bstage():
        dump = jnp.full((LANES,), GATH_ROWS - 16, jnp.int32)
        for c in range(FLUSH // LANES):
            bstage_v[pl.ds(c * LANES, LANES)] = dump

    def process_table(idx_hbm, table_hbm, tail_hbm, gath_hbm):
        pltpu.sync_copy(idx_hbm, idx_v)
        pltpu.sync_copy(tail_hbm, tail_v)

        def scan_chunk(c, ptr):
            r = idx_v[pl.ds(c * LANES, LANES)]
            m = jnp.logical_and(r >= lo, r < hi)
            plsc.store_compressed(b_v.at[pl.ds(ptr, LANES)],
                                  c * LANES + lane, mask=m)
            cnt = plsc.all_reduce_population_count(m)
            return ptr + cnt[0]

        nhits = lax.fori_loop(0, BATCH // LANES, scan_chunk, 0)
        nhchunks = (nhits + LANES - 1) // LANES

        def wstart(j):
            return pl.multiple_of(
                jnp.minimum(lo + j * WIN, MAX_WSTART), 128)

        def fire(j, slot):
            pltpu.async_copy(table_hbm.at[:, pl.ds(wstart(j), WIN)],
                             win_v.at[slot], sem.at[slot])

        def drain(slot):
            pltpu.make_async_copy(table_hbm.at[:, pl.ds(0, WIN)],
                                  win_v.at[slot], sem.at[slot]).wait()

        def flush():
            pltpu.async_copy(stage_v, gath_hbm.at[bstage_v], ssem)
            pltpu.make_async_copy(stage_v, gath_hbm.at[bstage_v],
                                  ssem).wait()
            reset_bstage()

        def window(j, carry):
            nstaged = carry
            slot = j % 2

            @pl.when(j + 1 < NWIN)
            def _():
                fire(j + 1, 1 - slot)

            drain(slot)
            wlo = lo + j * WIN
            whi = jnp.minimum(jnp.minimum(wlo + WIN, hi), ALIGNED_END)
            ws = wstart(j)

            def filt(c, wptr):
                bvec = b_v[pl.ds(c * LANES, LANES)]
                bsafe = lax.bitwise_and(bvec, BATCH - 1)
                r = plsc.load_gather(idx_v, [bsafe])
                m = ((c * LANES + lane) < nhits) & (r >= wlo) & (r < whi)
                plsc.store_compressed(wb_v.at[pl.ds(wptr, LANES)], bvec, mask=m)
                cnt = plsc.all_reduce_population_count(m)
                return wptr + cnt[0]

            wcount = lax.fori_loop(0, nhchunks, filt, 0)

            def group(g, nstaged2):
                gmask = (g * LANES + lane) < wcount
                bvec = wb_v[pl.ds(g * LANES, LANES)]
                bsafe = lax.bitwise_and(bvec, BATCH - 1)
                r = plsc.load_gather(idx_v, [bsafe])
                rl = r - ws
                slot_v = jnp.full((LANES,), slot, jnp.int32)
                srow = nstaged2 + lane
                for d in range(LATENT):
                    dv = jnp.full((LANES,), d, jnp.int32)
                    vals = plsc.load_gather(win_v, [slot_v, dv, rl],
                                            mask=gmask)
                    plsc.store_scatter(stage_v, [srow, dv], vals, mask=gmask)
                bmasked = jnp.where(gmask, bvec, GATH_ROWS - 16)
                bstage_v[pl.ds(nstaged2, LANES)] = bmasked
                nxt = nstaged2 + LANES

                @pl.when(nxt > FLUSH - LANES)
                def _():
                    flush()

                return jnp.where(nxt > FLUSH - LANES, 0, nxt)

            ngroups = (wcount + LANES - 1) // LANES
            return lax.fori_loop(0, ngroups, group, nstaged)

        fire(0, 0)
        nstaged = lax.fori_loop(0, NWIN, window, 0)

        def tfilt(c, wptr):
            bvec = b_v[pl.ds(c * LANES, LANES)]
            bsafe = lax.bitwise_and(bvec, BATCH - 1)
            r = plsc.load_gather(idx_v, [bsafe])
            m = ((c * LANES + lane) < nhits) & (r >= ALIGNED_END)
            plsc.store_compressed(wb_v.at[pl.ds(wptr, LANES)], bvec, mask=m)
            cnt = plsc.all_reduce_population_count(m)
            return wptr + cnt[0]

        tcount = lax.fori_loop(0, nhchunks, tfilt, 0)

        def tgroup(g, nstaged2):
            gmask = (g * LANES + lane) < tcount
            bvec = wb_v[pl.ds(g * LANES, LANES)]
            bsafe = lax.bitwise_and(bvec, BATCH - 1)
            r = plsc.load_gather(idx_v, [bsafe])
            rl = r - ALIGNED_END
            srow = nstaged2 + lane
            for d in range(LATENT):
                dv = jnp.full((LANES,), d, jnp.int32)
                vals = plsc.load_gather(tail_v, [dv, rl], mask=gmask)
                plsc.store_scatter(stage_v, [srow, dv], vals, mask=gmask)
            bmasked = jnp.where(gmask, bvec, GATH_ROWS - 16)
            bstage_v[pl.ds(nstaged2, LANES)] = bmasked
            nxt = nstaged2 + LANES

            @pl.when(nxt > FLUSH - LANES)
            def _():
                flush()

            return jnp.where(nxt > FLUSH - LANES, 0, nxt)

        tgroups = (tcount + LANES - 1) // LANES
        nstaged = lax.fori_loop(0, tgroups, tgroup, nstaged)

        @pl.when(nstaged > 0)
        def _():
            flush()

    reset_bstage()
    process_table(idxu_hbm, user_t_hbm, utail_hbm, ugath_hbm)
    process_table(idxi_hbm, item_t_hbm, itail_hbm, vgath_hbm)


def _dot_body(ugath_hbm, vgath_hbm, out_hbm, ublk_v, vblk_v, out_v):
    wid = lax.axis_index("s") * NUM_CORES + lax.axis_index("c")
    base = wid * B_PER_W

    lane = lax.iota(jnp.int32, LANES)

    for s in range(B_PER_W // 128):
        pltpu.sync_copy(
            ugath_hbm.at[pl.ds(base + s * 128, 128), :], ublk_v)
        pltpu.sync_copy(
            vgath_hbm.at[pl.ds(base + s * 128, 128), :], vblk_v)
        for c in range(128 // LANES):
            rows = c * LANES + lane
            acc = jnp.zeros((LANES,), jnp.float32)
            for d in range(LATENT):
                dv = jnp.full((LANES,), d, jnp.int32)
                u = plsc.load_gather(ublk_v, [rows, dv])
                v = plsc.load_gather(vblk_v, [rows, dv])
                acc = acc + u * v
            acc = jnp.clip(acc, 0.0, 5.0)
            out_v[pl.ds((s * 8 + c) * LANES, LANES)] = acc

    pltpu.sync_copy(out_v, out_hbm.at[pl.ds(base, B_PER_W)])


@jax.jit
def kernel(batched_inputs, user_hidden_emb, item_hidden_emb):
    idx_user = batched_inputs[:, 0].astype(jnp.int32)
    idx_item = batched_inputs[:, 2].astype(jnp.int32)
    utail = user_hidden_emb[ALIGNED_END:, :].T
    itail = item_hidden_emb[ALIGNED_END:, :].T
    utail = jnp.pad(utail, ((0, 0), (0, TAIL - (TABLE - ALIGNED_END))))
    itail = jnp.pad(itail, ((0, 0), (0, TAIL - (TABLE - ALIGNED_END))))
    mesh = plsc.VectorSubcoreMesh(core_axis_name="c", subcore_axis_name="s")
    params = pltpu.CompilerParams(
        needs_layout_passes=False, use_tc_tiling_on_sc=True)

    gather = pl.kernel(
        _gather_body,
        out_type=(
            jax.ShapeDtypeStruct((GATH_ROWS, 128), jnp.float32),
            jax.ShapeDtypeStruct((GATH_ROWS, 128), jnp.float32),
        ),
        mesh=mesh,
        scratch_types=[
            pltpu.VMEM((BATCH,), jnp.int32),
            pltpu.VMEM((BATCH,), jnp.int32),
            pltpu.VMEM((BATCH,), jnp.int32),
            pltpu.VMEM((2, LATENT, WIN), jnp.float32),
            pltpu.VMEM((LATENT, TAIL), jnp.float32),
            pltpu.VMEM((FLUSH, 128), jnp.float32),
            pltpu.VMEM((FLUSH,), jnp.int32),
            pltpu.SemaphoreType.DMA((2,)),
            pltpu.SemaphoreType.DMA,
        ],
        compiler_params=params,
    )
    ugath, vgath = gather(idx_user, idx_item,
                          user_hidden_emb.T, item_hidden_emb.T,
                          utail, itail)

    dot = pl.kernel(
        _dot_body,
        out_type=jax.ShapeDtypeStruct((BATCH,), jnp.float32),
        mesh=mesh,
        scratch_types=[
            pltpu.VMEM((128, 128), jnp.float32),
            pltpu.VMEM((128, 128), jnp.float32),
            pltpu.VMEM((B_PER_W,), jnp.float32),
        ],
        compiler_params=params,
    )
    return dot(ugath, vgath)


# speedup vs baseline: 6.8310x; 6.8310x over previous
import jax
import jax.numpy as jnp
from jax import lax
from jax.experimental import pallas as pl
from jax.experimental.pallas import tpu as pltpu
from jax.experimental.pallas import tpu_sc as plsc

LATENT = 32
BATCH = 16384
TABLE = 1000000

NUM_CORES = 2
NUM_SUBCORES = 16
LANES = 16
NW = NUM_CORES * NUM_SUBCORES
B_PER_W = BATCH // NW

TILECOLS = TABLE // 128
R_PER_W = ((TILECOLS + NW - 1) // NW) * 128
WIN = 512
NWIN = (R_PER_W + WIN - 1) // WIN
ALIGNED_END = TILECOLS * 128
MAX_WSTART = ALIGNED_END - WIN
TAIL = 128
GATH_ROWS = BATCH + NW * 16
FLUSH = 64


def _gather_body(idxu_hbm, idxi_hbm, user_t_hbm, item_t_hbm,
                 utail_hbm, itail_hbm, ugath_hbm, vgath_hbm,
                 idx_v, b_v, wb_v, win_v, tail_v, stage_v, bstage_v,
                 sem, ssem):
    wid = lax.axis_index("s") * NUM_CORES + lax.axis_index("c")
    lo = wid * R_PER_W
    hi = jnp.minimum(lo + R_PER_W, TABLE)

    lane = lax.iota(jnp.int32, LANES)
    rows_lo = lane
    rows_hi = lane + LANES

    def reset_bstage():
        dump = BATCH + wid * LANES + lane
        for c in range(FLUSH // LANES):
            bstage_v[pl.ds(c * LANES, LANES)] = dump

    def process_table(idx_hbm, table_hbm, tail_hbm, gath_hbm):
        pltpu.sync_copy(idx_hbm, idx_v)
        pltpu.sync_copy(tail_hbm, tail_v)

        def scan_chunk(c, ptr):
            r = idx_v[pl.ds(c * LANES, LANES)]
            m = jnp.logical_and(r >= lo, r < hi)
            plsc.store_compressed(b_v.at[pl.ds(ptr, LANES)],
                                  c * LANES + lane, mask=m)
            cnt = plsc.all_reduce_population_count(m)
            return ptr + cnt[0]

        nhits = lax.fori_loop(0, BATCH // LANES, scan_chunk, 0)
        nhchunks = (nhits + LANES - 1) // LANES

        def wstart(j):
            return pl.multiple_of(
                jnp.minimum(lo + j * WIN, MAX_WSTART), 128)

        def fire(j, slot):
            pltpu.async_copy(table_hbm.at[:, pl.ds(wstart(j), WIN)],
                             win_v.at[slot], sem.at[slot])

        def drain(slot):
            pltpu.make_async_copy(table_hbm.at[:, pl.ds(0, WIN)],
                                  win_v.at[slot], sem.at[slot]).wait()

        def flush():
            pltpu.async_copy(stage_v, gath_hbm.at[bstage_v], ssem)
            pltpu.make_async_copy(stage_v, gath_hbm.at[bstage_v],
                                  ssem).wait()
            reset_bstage()

        def window(j, carry):
            nstaged = carry
            slot = j % 2

            @pl.when(j + 1 < NWIN)
            def _():
                fire(j + 1, 1 - slot)

            drain(slot)
            wlo = lo + j * WIN
            whi = jnp.minimum(jnp.minimum(wlo + WIN, hi), ALIGNED_END)
            ws = wstart(j)

            def filt(c, wptr):
                bvec = b_v[pl.ds(c * LANES, LANES)]
                bsafe = lax.bitwise_and(bvec, BATCH - 1)
                r = plsc.load_gather(idx_v, [bsafe])
                m = ((c * LANES + lane) < nhits) & (r >= wlo) & (r < whi)
                plsc.store_compressed(wb_v.at[pl.ds(wptr, LANES)], bvec, mask=m)
                cnt = plsc.all_reduce_population_count(m)
                return wptr + cnt[0]

            wcount = lax.fori_loop(0, nhchunks, filt, 0)

            def group(g, nstaged2):
                gmask = (g * LANES + lane) < wcount
                bvec = wb_v[pl.ds(g * LANES, LANES)]
                bsafe = lax.bitwise_and(bvec, BATCH - 1)
                r = plsc.load_gather(idx_v, [bsafe])
                rl = r - ws
                slot_v = jnp.full((LANES,), slot, jnp.int32)
                srow = nstaged2 + lane
                for d in range(LATENT):
                    dv = jnp.full((LANES,), d, jnp.int32)
                    vals = plsc.load_gather(win_v, [slot_v, dv, rl],
                                            mask=gmask)
                    plsc.store_scatter(stage_v, [srow, dv], vals, mask=gmask)
                bmasked = jnp.where(gmask, bvec, BATCH + wid * LANES + lane)
                bstage_v[pl.ds(nstaged2, LANES)] = bmasked
                nxt = nstaged2 + LANES

                @pl.when(nxt > FLUSH - LANES)
                def _():
                    flush()

                return jnp.where(nxt > FLUSH - LANES, 0, nxt)

            ngroups = (wcount + LANES - 1) // LANES
            return lax.fori_loop(0, ngroups, group, nstaged)

        fire(0, 0)
        nstaged = lax.fori_loop(0, NWIN, window, 0)

        def tfilt(c, wptr):
            bvec = b_v[pl.ds(c * LANES, LANES)]
            bsafe = lax.bitwise_and(bvec, BATCH - 1)
            r = plsc.load_gather(idx_v, [bsafe])
            m = ((c * LANES + lane) < nhits) & (r >= ALIGNED_END)
            plsc.store_compressed(wb_v.at[pl.ds(wptr, LANES)], bvec, mask=m)
            cnt = plsc.all_reduce_population_count(m)
            return wptr + cnt[0]

        tcount = lax.fori_loop(0, nhchunks, tfilt, 0)

        def tgroup(g, nstaged2):
            gmask = (g * LANES + lane) < tcount
            bvec = wb_v[pl.ds(g * LANES, LANES)]
            bsafe = lax.bitwise_and(bvec, BATCH - 1)
            r = plsc.load_gather(idx_v, [bsafe])
            rl = r - ALIGNED_END
            srow = nstaged2 + lane
            for d in range(LATENT):
                dv = jnp.full((LANES,), d, jnp.int32)
                vals = plsc.load_gather(tail_v, [dv, rl], mask=gmask)
                plsc.store_scatter(stage_v, [srow, dv], vals, mask=gmask)
            bmasked = jnp.where(gmask, bvec, BATCH + wid * LANES + lane)
            bstage_v[pl.ds(nstaged2, LANES)] = bmasked
            nxt = nstaged2 + LANES

            @pl.when(nxt > FLUSH - LANES)
            def _():
                flush()

            return jnp.where(nxt > FLUSH - LANES, 0, nxt)

        tgroups = (tcount + LANES - 1) // LANES
        nstaged = lax.fori_loop(0, tgroups, tgroup, nstaged)

        @pl.when(nstaged > 0)
        def _():
            flush()

    reset_bstage()
    process_table(idxu_hbm, user_t_hbm, utail_hbm, ugath_hbm)
    process_table(idxi_hbm, item_t_hbm, itail_hbm, vgath_hbm)


def _dot_body(ugath_hbm, vgath_hbm, out_hbm, ublk_v, vblk_v, out_v):
    wid = lax.axis_index("s") * NUM_CORES + lax.axis_index("c")
    base = wid * B_PER_W

    lane = lax.iota(jnp.int32, LANES)

    for s in range(B_PER_W // 128):
        pltpu.sync_copy(
            ugath_hbm.at[pl.ds(base + s * 128, 128), :], ublk_v)
        pltpu.sync_copy(
            vgath_hbm.at[pl.ds(base + s * 128, 128), :], vblk_v)
        for c in range(128 // LANES):
            rows = c * LANES + lane
            acc = jnp.zeros((LANES,), jnp.float32)
            for d in range(LATENT):
                dv = jnp.full((LANES,), d, jnp.int32)
                u = plsc.load_gather(ublk_v, [rows, dv])
                v = plsc.load_gather(vblk_v, [rows, dv])
                acc = acc + u * v
            acc = jnp.clip(acc, 0.0, 5.0)
            out_v[pl.ds((s * 8 + c) * LANES, LANES)] = acc

    pltpu.sync_copy(out_v, out_hbm.at[pl.ds(base, B_PER_W)])


@jax.jit
def kernel(batched_inputs, user_hidden_emb, item_hidden_emb):
    idx_user = batched_inputs[:, 0].astype(jnp.int32)
    idx_item = batched_inputs[:, 2].astype(jnp.int32)
    utail = user_hidden_emb[ALIGNED_END:, :].T
    itail = item_hidden_emb[ALIGNED_END:, :].T
    utail = jnp.pad(utail, ((0, 0), (0, TAIL - (TABLE - ALIGNED_END))))
    itail = jnp.pad(itail, ((0, 0), (0, TAIL - (TABLE - ALIGNED_END))))
    mesh = plsc.VectorSubcoreMesh(core_axis_name="c", subcore_axis_name="s")
    params = pltpu.CompilerParams(
        needs_layout_passes=False, use_tc_tiling_on_sc=True)

    gather = pl.kernel(
        _gather_body,
        out_type=(
            jax.ShapeDtypeStruct((GATH_ROWS, 128), jnp.float32),
            jax.ShapeDtypeStruct((GATH_ROWS, 128), jnp.float32),
        ),
        mesh=mesh,
        scratch_types=[
            pltpu.VMEM((BATCH,), jnp.int32),
            pltpu.VMEM((BATCH,), jnp.int32),
            pltpu.VMEM((BATCH,), jnp.int32),
            pltpu.VMEM((2, LATENT, WIN), jnp.float32),
            pltpu.VMEM((LATENT, TAIL), jnp.float32),
            pltpu.VMEM((FLUSH, 128), jnp.float32),
            pltpu.VMEM((FLUSH,), jnp.int32),
            pltpu.SemaphoreType.DMA((2,)),
            pltpu.SemaphoreType.DMA,
        ],
        compiler_params=params,
    )
    ugath, vgath = gather(idx_user, idx_item,
                          user_hidden_emb.T, item_hidden_emb.T,
                          utail, itail)

    dot = pl.kernel(
        _dot_body,
        out_type=jax.ShapeDtypeStruct((BATCH,), jnp.float32),
        mesh=mesh,
        scratch_types=[
            pltpu.VMEM((128, 128), jnp.float32),
            pltpu.VMEM((128, 128), jnp.float32),
            pltpu.VMEM((B_PER_W,), jnp.float32),
        ],
        compiler_params=params,
    )
    return dot(ugath, vgath)


# speedup vs baseline: 7.1328x; 1.0442x over previous
import jax
import jax.numpy as jnp
from jax import lax
from jax.experimental import pallas as pl
from jax.experimental.pallas import tpu as pltpu
from jax.experimental.pallas import tpu_sc as plsc

LATENT = 32
BATCH = 16384

NUM_CORES = 2
NUM_SUBCORES = 16
LANES = 16
NW = NUM_CORES * NUM_SUBCORES
B_PER_W = BATCH // NW
CHUNKS = B_PER_W // LANES
NBUF = 8


def _cf_body(idxu_hbm, idxi_hbm, user_t_hbm, item_t_hbm, out_hbm,
             idxu_v, idxi_v, ublk_v, vblk_v, out_v, usems, vsems):
    wid = lax.axis_index("s") * NUM_CORES + lax.axis_index("c")
    base = wid * B_PER_W

    pltpu.sync_copy(idxu_hbm.at[pl.ds(base, B_PER_W)], idxu_v)
    pltpu.sync_copy(idxi_hbm.at[pl.ds(base, B_PER_W)], idxi_v)

    lane = lax.iota(jnp.int32, LANES)
    rows_lo = lane
    rows_hi = lane + LANES

    def fire(slot, cu, ci):
        off_u = pl.multiple_of(cu * 128, 128)
        off_i = pl.multiple_of(ci * 128, 128)
        pltpu.async_copy(
            user_t_hbm.at[:, pl.ds(off_u, 128)], ublk_v.at[slot],
            usems.at[slot])
        pltpu.async_copy(
            item_t_hbm.at[:, pl.ds(off_i, 128)], vblk_v.at[slot],
            vsems.at[slot])

    def drain(slot):
        pltpu.make_async_copy(
            user_t_hbm.at[:, pl.ds(0, 128)], ublk_v.at[slot],
            usems.at[slot]).wait()
        pltpu.make_async_copy(
            item_t_hbm.at[:, pl.ds(0, 128)], vblk_v.at[slot],
            vsems.at[slot]).wait()

    u0 = idxu_v[pl.ds(0, LANES)]
    i0 = idxi_v[pl.ds(0, LANES)]
    cu0 = lax.shift_right_logical(u0, 7)
    ci0 = lax.shift_right_logical(i0, 7)
    for k in range(NBUF):
        fire(k, cu0[k], ci0[k])

    def step(c, carry):
        uvec = idxu_v[pl.ds(c * LANES, LANES)]
        ivec = idxi_v[pl.ds(c * LANES, LANES)]
        cu_vec = lax.shift_right_logical(uvec, 7)
        ci_vec = lax.shift_right_logical(ivec, 7)
        lu_vec = lax.bitwise_and(uvec, 127)
        li_vec = lax.bitwise_and(ivec, 127)
        cn = jnp.where(c + 1 < CHUNKS, c + 1, 0)
        nuvec = idxu_v[pl.ds(cn * LANES, LANES)]
        nivec = idxi_v[pl.ds(cn * LANES, LANES)]
        pu_vec = lax.shift_right_logical(nuvec, 7)
        pi_vec = lax.shift_right_logical(nivec, 7)

        acc = jnp.zeros((LANES,), jnp.float32)
        for k in range(LANES):
            slot = k % NBUF
            drain(slot)
            slot_v = jnp.full((LANES,), slot, jnp.int32)
            lu = jnp.full((LANES,), lu_vec[k], jnp.int32)
            li = jnp.full((LANES,), li_vec[k], jnp.int32)
            u_lo = plsc.load_gather(ublk_v, [slot_v, rows_lo, lu])
            u_hi = plsc.load_gather(ublk_v, [slot_v, rows_hi, lu])
            v_lo = plsc.load_gather(vblk_v, [slot_v, rows_lo, li])
            v_hi = plsc.load_gather(vblk_v, [slot_v, rows_hi, li])
            p = u_lo * v_lo + u_hi * v_hi
            s = jnp.sum(p)
            acc = jnp.where(lane == k, s, acc)
            if k < NBUF:
                fire(slot, cu_vec[k + NBUF], ci_vec[k + NBUF])
            else:
                @pl.when(c + 1 < CHUNKS)
                def _():
                    fire(slot, pu_vec[k - NBUF], pi_vec[k - NBUF])
        acc = jnp.clip(acc, 0.0, 5.0)
        out_v[pl.ds(c * LANES, LANES)] = acc
        return carry

    lax.fori_loop(0, CHUNKS, step, 0)

    pltpu.sync_copy(out_v, out_hbm.at[pl.ds(base, B_PER_W)])


@jax.jit
def kernel(batched_inputs, user_hidden_emb, item_hidden_emb):
    idx_user = batched_inputs[:, 0].astype(jnp.int32)
    idx_item = batched_inputs[:, 2].astype(jnp.int32)
    mesh = plsc.VectorSubcoreMesh(core_axis_name="c", subcore_axis_name="s")
    run = pl.kernel(
        _cf_body,
        out_type=jax.ShapeDtypeStruct((BATCH,), jnp.float32),
        mesh=mesh,
        scratch_types=[
            pltpu.VMEM((B_PER_W,), jnp.int32),
            pltpu.VMEM((B_PER_W,), jnp.int32),
            pltpu.VMEM((NBUF, LATENT, 128), jnp.float32),
            pltpu.VMEM((NBUF, LATENT, 128), jnp.float32),
            pltpu.VMEM((B_PER_W,), jnp.float32),
            pltpu.SemaphoreType.DMA((NBUF,)),
            pltpu.SemaphoreType.DMA((NBUF,)),
        ],
        compiler_params=pltpu.CompilerParams(
            needs_layout_passes=False, use_tc_tiling_on_sc=True),
    )
    return run(idx_user, idx_item, user_hidden_emb.T, item_hidden_emb.T)
